# baseline (speedup 1.0000x reference)
import functools

import jax
import jax.numpy as jnp
from jax import lax
from jax.experimental import pallas as pl
from jax.experimental.pallas import tpu as pltpu
from jax.experimental.pallas import tpu_sc as plsc

NC, NS = 2, 16
NW = NC * NS
BATCH, N_FIELDS = 16384, 26
D = 64
BR_W = BATCH // NW
GB = 32
NGROUP = BR_W // GB
NBUF = 2

_mesh = plsc.VectorSubcoreMesh(
    core_axis_name="c", subcore_axis_name="s", num_cores=NC, num_subcores=NS)


@functools.partial(
    pl.kernel,
    out_type=jax.ShapeDtypeStruct((BATCH, N_FIELDS, D), jnp.float32),
    mesh=_mesh,
    scratch_types=[
        pltpu.VMEM((BR_W, N_FIELDS), jnp.int32),
        pltpu.VMEM((GB, N_FIELDS, D), jnp.float32),
        pltpu.VMEM((GB, N_FIELDS, D), jnp.float32),
        pltpu.SemaphoreType.DMA,
        pltpu.SemaphoreType.DMA,
        pltpu.SemaphoreType.DMA,
        pltpu.SemaphoreType.DMA,
    ],
    compiler_params=pltpu.CompilerParams(use_tc_tiling_on_sc=False),
)
def _gather(ids_hbm, table_hbm, out_hbm, idx_v, rows0, rows1,
            gs0, gs1, ws0, ws1):
    wid = lax.axis_index("s") * NC + lax.axis_index("c")
    base = wid * BR_W
    pltpu.sync_copy(ids_hbm.at[pl.ds(base, BR_W)], idx_v)

    rows = (rows0, rows1)
    gsem = (gs0, gs1)
    wsem = (ws0, ws1)

    def fire_g(g, s):
        @pl.loop(0, GB)
        def _row(j):
            pltpu.async_copy(table_hbm.at[idx_v.at[g * GB + j]],
                             rows[s].at[j], gsem[s])

    def drain_g(s):
        pltpu.make_async_copy(
            out_hbm.at[pl.ds(0, GB)], rows[s], gsem[s]).wait()

    def fire_w(g, s):
        pltpu.async_copy(rows[s], out_hbm.at[pl.ds(base + g * GB, GB)],
                         wsem[s])

    def drain_w(g, s):
        pltpu.make_async_copy(
            rows[s], out_hbm.at[pl.ds(base + g * GB, GB)], wsem[s]).wait()

    fire_g(0, 0)
    for g in range(NGROUP):
        gf = g + 1
        if gf < NGROUP:
            if gf >= NBUF:
                drain_w(gf - NBUF, gf % NBUF)
            fire_g(gf, gf % NBUF)
        s = g % NBUF
        drain_g(s)
        fire_w(g, s)
    for g in range(NGROUP - NBUF, NGROUP):
        drain_w(g, g % NBUF)


def kernel(ids, table):
    return _gather(jnp.asarray(ids, jnp.int32), table)


# speedup vs baseline: 1.0715x; 1.0715x over previous
import functools

import jax
import jax.numpy as jnp
from jax import lax
from jax.experimental import pallas as pl
from jax.experimental.pallas import tpu as pltpu
from jax.experimental.pallas import tpu_sc as plsc

NC, NS = 2, 16
NW = NC * NS
BATCH, N_FIELDS = 16384, 26
D = 64
NUM_ROWS = 1000000
BR_W = BATCH // NW
GB = 32
NGROUP = BR_W // GB
NBUF = 2

_mesh = plsc.VectorSubcoreMesh(
    core_axis_name="c", subcore_axis_name="s", num_cores=NC, num_subcores=NS)


@functools.partial(
    pl.kernel,
    out_type=jax.ShapeDtypeStruct((BATCH, N_FIELDS, D), jnp.float32),
    mesh=_mesh,
    scratch_types=[
        pltpu.VMEM((BR_W, N_FIELDS), jnp.int32),
        pltpu.VMEM((GB, N_FIELDS, D), jnp.float32),
        pltpu.VMEM((GB, N_FIELDS, D), jnp.float32),
        pltpu.SemaphoreType.DMA,
        pltpu.SemaphoreType.DMA,
        pltpu.SemaphoreType.DMA,
        pltpu.SemaphoreType.DMA,
    ],
    compiler_params=pltpu.CompilerParams(use_tc_tiling_on_sc=False),
)
def _gather(ids_hbm, table_hbm, out_hbm, idx_v, rows0, rows1,
            gs0, gs1, ws0, ws1):
    wid = lax.axis_index("s") * NC + lax.axis_index("c")
    base = wid * BR_W
    pltpu.sync_copy(ids_hbm.at[pl.ds(base, BR_W)], idx_v)

    rows = (rows0, rows1)
    gsem = (gs0, gs1)
    wsem = (ws0, ws1)

    def fire_g(g, s):
        @pl.loop(0, GB)
        def _row(j):
            pltpu.async_copy(table_hbm.at[idx_v.at[g * GB + j]],
                             rows[s].at[j], gsem[s])

    def drain_g(s):
        pltpu.make_async_copy(
            out_hbm.at[pl.ds(0, GB)], rows[s], gsem[s]).wait()

    def fire_w(g, s):
        pltpu.async_copy(rows[s], out_hbm.at[pl.ds(base + g * GB, GB)],
                         wsem[s])

    def drain_w(g, s):
        pltpu.make_async_copy(
            rows[s], out_hbm.at[pl.ds(base + g * GB, GB)], wsem[s]).wait()

    fire_g(0, 0)
    for g in range(NGROUP):
        gf = g + 1
        if gf < NGROUP:
            if gf >= NBUF:
                drain_w(gf - NBUF, gf % NBUF)
            fire_g(gf, gf % NBUF)
        s = g % NBUF
        drain_g(s)
        fire_w(g, s)
    for g in range(NGROUP - NBUF, NGROUP):
        drain_w(g, g % NBUF)


def kernel(ids, table):
    table2 = jnp.pad(table, ((0, 0), (0, D))).reshape(2 * NUM_ROWS, D)
    ids2 = jnp.asarray(ids, jnp.int32) * 2
    return _gather(ids2, table2)


# speedup vs baseline: 1.3395x; 1.2501x over previous
import functools

import jax
import jax.numpy as jnp
from jax import lax
from jax.experimental import pallas as pl
from jax.experimental.pallas import tpu as pltpu
from jax.experimental.pallas import tpu_sc as plsc

NC, NS = 2, 16
NW = NC * NS
BATCH, N_FIELDS = 16384, 26
D = 64
NUM_ROWS = 1000000
BR_W = BATCH // NW
GB = 32
NGROUP = BR_W // GB
NBUF = 2

_mesh = plsc.VectorSubcoreMesh(
    core_axis_name="c", subcore_axis_name="s", num_cores=NC, num_subcores=NS)


@functools.partial(
    pl.kernel,
    out_type=jax.ShapeDtypeStruct((BATCH, 32, 128), jnp.float32),
    mesh=_mesh,
    scratch_types=[
        pltpu.VMEM((BR_W, N_FIELDS), jnp.int32),
        pltpu.VMEM((GB, N_FIELDS, D), jnp.float32),
        pltpu.VMEM((GB, N_FIELDS, D), jnp.float32),
        pltpu.SemaphoreType.DMA,
        pltpu.SemaphoreType.DMA,
        pltpu.SemaphoreType.DMA,
        pltpu.SemaphoreType.DMA,
    ],
    compiler_params=pltpu.CompilerParams(use_tc_tiling_on_sc=False),
)
def _gather(ids_hbm, table_hbm, out_hbm, idx_v, rows0, rows1,
            gs0, gs1, ws0, ws1):
    wid = lax.axis_index("s") * NC + lax.axis_index("c")
    base = wid * BR_W
    pltpu.sync_copy(ids_hbm.at[pl.ds(base, BR_W)], idx_v)

    rows = (rows0, rows1)
    gsem = (gs0, gs1)
    wsem = (ws0, ws1)

    def fire_g(g, s):
        @pl.loop(0, GB)
        def _row(j):
            pltpu.async_copy(table_hbm.at[idx_v.at[g * GB + j]],
                             rows[s].at[j], gsem[s])

    def drain_g(s):
        pltpu.make_async_copy(
            out_hbm.at[pl.ds(0, GB), pl.ds(0, N_FIELDS), pl.ds(0, D)],
            rows[s], gsem[s]).wait()

    def _wslice(g):
        return out_hbm.at[pl.ds(base + g * GB, GB), pl.ds(0, N_FIELDS),
                          pl.ds(0, D)]

    def fire_w(g, s):
        pltpu.async_copy(rows[s], _wslice(g), wsem[s])

    def drain_w(g, s):
        pltpu.make_async_copy(rows[s], _wslice(g), wsem[s]).wait()

    fire_g(0, 0)
    for g in range(NGROUP):
        gf = g + 1
        if gf < NGROUP:
            if gf >= NBUF:
                drain_w(gf - NBUF, gf % NBUF)
            fire_g(gf, gf % NBUF)
        s = g % NBUF
        drain_g(s)
        fire_w(g, s)
    for g in range(NGROUP - NBUF, NGROUP):
        drain_w(g, g % NBUF)


def kernel(ids, table):
    table2 = jnp.pad(table, ((0, 0), (0, D))).reshape(2 * NUM_ROWS, D)
    ids2 = jnp.asarray(ids, jnp.int32) * 2
    out_big = _gather(ids2, table2)
    return out_big[:, :N_FIELDS, :D]
